# baseline (speedup 1.0000x reference)
import jax
import jax.numpy as jnp
from jax.experimental import pallas as pl

_GRID = 10
_X_ROWS = 10000 // _GRID
_E_ROWS = (320000 * 16 // 128) // _GRID


def _copy_body(x_ref, e_ref, ox_ref, oe_ref):
    ox_ref[...] = x_ref[...]
    oe_ref[...] = e_ref[...]


def kernel(x, edge_index, edge_attr):
    del edge_index
    n_edges, d_edge = edge_attr.shape
    e2 = edge_attr.reshape(n_edges * d_edge // 128, 128)
    out_x, out_e = pl.pallas_call(
        _copy_body,
        grid=(_GRID,),
        in_specs=[
            pl.BlockSpec((_X_ROWS, 128), lambda i: (i, 0)),
            pl.BlockSpec((_E_ROWS, 128), lambda i: (i, 0)),
        ],
        out_specs=[
            pl.BlockSpec((_X_ROWS, 128), lambda i: (i, 0)),
            pl.BlockSpec((_E_ROWS, 128), lambda i: (i, 0)),
        ],
        out_shape=[
            jax.ShapeDtypeStruct(x.shape, x.dtype),
            jax.ShapeDtypeStruct(e2.shape, e2.dtype),
        ],
    )(x, e2)
    return (out_x, out_e.reshape(n_edges, d_edge))
